# Initial kernel scaffold; baseline (speedup 1.0000x reference)
import jax
import jax.numpy as jnp
from jax.experimental import pallas as pl


def kernel(x, edge_index, W1, att_src1, att_dst1, b1, W2, att_src2, att_dst2, b2, Wh, bh):
    raise NotImplementedError("write your pallas kernel here")


# speedup vs baseline: 30.3485x; 30.3485x over previous
import functools

import jax
import jax.numpy as jnp
from jax import lax
from jax.experimental import pallas as pl
from jax.experimental.pallas import tpu as pltpu
from jax.experimental.pallas import tpu_sc as plsc

N = 10000
NP = 10240
D = 128
E = 320000
E_TOT = N + E
KCH = 88
NTILES = 32
TCH = 120
E_PAD = NTILES * TCH * KCH
PAD_NODE = 10200
BLK = 1024
ROWS_PER_SUB = NP // 16


def _lgather(v, idx):
    dn = lax.GatherDimensionNumbers(
        offset_dims=(), collapsed_slice_dims=(0,), start_index_map=(0,))
    return lax.gather(v, idx[:, None], dn, (1,),
                      mode=lax.GatherScatterMode.PROMISE_IN_BOUNDS)


def _elu(v):
    return jnp.where(v > 0, v, jnp.exp(jnp.minimum(v, 0.0)) - 1.0)


def _tc1_body(x_r, w_r, a_r, h_r, att_r, m_r, macc):
    i = pl.program_id(0)
    n = pl.num_programs(0)
    h = jnp.dot(x_r[...], w_r[...], preferred_element_type=jnp.float32)
    h_r[...] = h
    att = jnp.dot(h, a_r[...], preferred_element_type=jnp.float32)
    att_r[...] = att
    m = jnp.max(att, axis=0, keepdims=True)
    macc[...] = jnp.where(i == 0, m, jnp.maximum(macc[...], m))

    @pl.when(i == n - 1)
    def _():
        m_r[...] = macc[...]


def _tc1(x_p, W1, amat1):
    return pl.pallas_call(
        _tc1_body,
        grid=(NP // BLK,),
        in_specs=[
            pl.BlockSpec((BLK, D), lambda i: (i, 0)),
            pl.BlockSpec((D, D), lambda i: (0, 0)),
            pl.BlockSpec((D, 16), lambda i: (0, 0)),
        ],
        out_specs=[
            pl.BlockSpec((BLK, D), lambda i: (i, 0)),
            pl.BlockSpec((BLK, 16), lambda i: (i, 0)),
            pl.BlockSpec((1, 16), lambda i: (0, 0)),
        ],
        out_shape=[
            jax.ShapeDtypeStruct((NP, D), jnp.float32),
            jax.ShapeDtypeStruct((NP, 16), jnp.float32),
            jax.ShapeDtypeStruct((1, 16), jnp.float32),
        ],
        scratch_shapes=[pltpu.VMEM((1, 16), jnp.float32)],
    )(x_p, W1, amat1)


def _make_sc_edge(C, H):
    CPH = C // H
    mesh = plsc.VectorSubcoreMesh(core_axis_name="c", subcore_axis_name="s")

    @functools.partial(
        pl.kernel,
        out_type=(
            jax.ShapeDtypeStruct((2, NP, C), jnp.float32),
            jax.ShapeDtypeStruct((2, NP, 16), jnp.float32),
        ),
        mesh=mesh,
        compiler_params=pltpu.CompilerParams(use_tc_tiling_on_sc=False),
        scratch_types=[
            pltpu.VMEM((TCH, KCH), jnp.int32),
            pltpu.VMEM((TCH, KCH), jnp.int32),
            pltpu.VMEM((KCH, C), jnp.float32),
            pltpu.VMEM((KCH, 16), jnp.float32),
            pltpu.VMEM((KCH, 16), jnp.float32),
            pltpu.VMEM((KCH, 16), jnp.float32),
            pltpu.VMEM((16,), jnp.float32),
            pltpu.VMEM_SHARED((NP, C), jnp.float32),
            pltpu.VMEM_SHARED((NP, 16), jnp.float32),
        ],
    )
    def k(src_hbm, dst_hbm, h_hbm, att_hbm, m_hbm, oraw, oden,
          src2d, dst2d, rows, A, B, sden, gv, sout, sdsh):
        cid = lax.axis_index("c")
        sid = lax.axis_index("s")
        wid = sid * 2 + cid
        c0 = wid * TCH
        pltpu.sync_copy(src_hbm.at[pl.ds(c0, TCH)], src2d)
        pltpu.sync_copy(dst_hbm.at[pl.ds(c0, TCH)], dst2d)
        pltpu.sync_copy(m_hbm.at[0], gv)

        iota = lax.iota(jnp.int32, 16)
        idxshift = (iota + 8) & 15
        maskH = jnp.where(iota < H, 1.0, 0.0).astype(jnp.float32)
        m = gv[...]
        g = jnp.maximum(m + _lgather(m, idxshift), 0.0) * maskH
        hsel = [jnp.full((16,), h, jnp.int32) for h in range(H)]
        z16 = jnp.zeros((16,), jnp.float32)

        def zrow(i, c):
            for kk in range(C // 16):
                rows[i, pl.ds(kk * 16, 16)] = z16
            sden[i] = z16
            return c
        lax.fori_loop(0, KCH, zrow, 0)
        r0 = sid * ROWS_PER_SUB
        for q in range(ROWS_PER_SUB // 80):
            pltpu.sync_copy(rows.at[pl.ds(0, 80)],
                            sout.at[pl.ds(r0 + q * 80, 80)])
            pltpu.sync_copy(sden.at[pl.ds(0, 80)],
                            sdsh.at[pl.ds(r0 + q * 80, 80)])
        plsc.subcore_barrier()

        def chunk(c, carry):
            pltpu.sync_copy(h_hbm.at[src2d.at[c]], rows)
            pltpu.sync_copy(att_hbm.at[src2d.at[c]], A)
            pltpu.sync_copy(att_hbm.at[dst2d.at[c]], B)

            def edge(e, cc):
                av = A[e]
                bv = B[e]
                al = av + _lgather(bv, idxshift)
                al = jnp.maximum(al, 0.2 * al)
                en = jnp.exp(al - g)
                sden[e] = en * maskH
                for h in range(H):
                    cs = _lgather(en, hsel[h])
                    for kk in range(CPH // 16):
                        sl = pl.ds(h * CPH + kk * 16, 16)
                        rows[e, sl] = rows[e, sl] * cs
                return cc
            lax.fori_loop(0, KCH, edge, 0)
            pltpu.sync_copy(rows, sout.at[dst2d.at[c]], add=True)
            pltpu.sync_copy(sden, sdsh.at[dst2d.at[c]], add=True)
            return carry
        lax.fori_loop(0, TCH, chunk, 0)
        plsc.subcore_barrier()

        pltpu.sync_copy(sout.at[pl.ds(r0, ROWS_PER_SUB)],
                        oraw.at[cid, pl.ds(r0, ROWS_PER_SUB)])
        pltpu.sync_copy(sdsh.at[pl.ds(r0, ROWS_PER_SUB)],
                        oden.at[cid, pl.ds(r0, ROWS_PER_SUB)])

    return k


_sc_cache = {}


def _sc_edge(C, H):
    if (C, H) not in _sc_cache:
        _sc_cache[(C, H)] = _make_sc_edge(C, H)
    return _sc_cache[(C, H)]


def _tc2_body(o_r, d_r, b_r, m1_r, w2_r, a2_r, h2_r, att_r, m_r, macc):
    i = pl.program_id(0)
    n = pl.num_programs(0)
    raw = o_r[0] + o_r[1]
    den = d_r[0] + d_r[1]
    denx = jnp.dot(den, m1_r[...], preferred_element_type=jnp.float32)
    x1 = _elu(raw / (denx + 1e-16) + b_r[...])
    h2 = jnp.dot(x1, w2_r[...], preferred_element_type=jnp.float32)
    h2_r[...] = h2
    att = jnp.dot(h2, a2_r[...], preferred_element_type=jnp.float32)
    att_r[...] = att
    m = jnp.max(att, axis=0, keepdims=True)
    macc[...] = jnp.where(i == 0, m, jnp.maximum(macc[...], m))

    @pl.when(i == n - 1)
    def _():
        m_r[...] = macc[...]


def _tc2(oraw, oden, b1r, M1, W2, amat2):
    return pl.pallas_call(
        _tc2_body,
        grid=(NP // BLK,),
        in_specs=[
            pl.BlockSpec((2, BLK, D), lambda i: (0, i, 0)),
            pl.BlockSpec((2, BLK, 16), lambda i: (0, i, 0)),
            pl.BlockSpec((1, D), lambda i: (0, 0)),
            pl.BlockSpec((16, D), lambda i: (0, 0)),
            pl.BlockSpec((D, 32), lambda i: (0, 0)),
            pl.BlockSpec((32, 16), lambda i: (0, 0)),
        ],
        out_specs=[
            pl.BlockSpec((BLK, 32), lambda i: (i, 0)),
            pl.BlockSpec((BLK, 16), lambda i: (i, 0)),
            pl.BlockSpec((1, 16), lambda i: (0, 0)),
        ],
        out_shape=[
            jax.ShapeDtypeStruct((NP, 32), jnp.float32),
            jax.ShapeDtypeStruct((NP, 16), jnp.float32),
            jax.ShapeDtypeStruct((1, 16), jnp.float32),
        ],
        scratch_shapes=[pltpu.VMEM((1, 16), jnp.float32)],
    )(oraw, oden, b1r, M1, W2, amat2)


def _tc3_body(o_r, d_r, b_r, m2_r, wh_r, bh_r, y_r):
    raw = o_r[0] + o_r[1]
    den = d_r[0] + d_r[1]
    denx = jnp.dot(den, m2_r[...], preferred_element_type=jnp.float32)
    x2 = _elu(raw / (denx + 1e-16) + b_r[...])
    y_r[...] = jnp.dot(x2, wh_r[...],
                       preferred_element_type=jnp.float32) + bh_r[...]


def _tc3(oraw2, oden2, b2r, M2, Wh, bhr):
    return pl.pallas_call(
        _tc3_body,
        grid=(NP // BLK,),
        in_specs=[
            pl.BlockSpec((2, BLK, 32), lambda i: (0, i, 0)),
            pl.BlockSpec((2, BLK, 16), lambda i: (0, i, 0)),
            pl.BlockSpec((1, 32), lambda i: (0, 0)),
            pl.BlockSpec((16, 32), lambda i: (0, 0)),
            pl.BlockSpec((32, 1), lambda i: (0, 0)),
            pl.BlockSpec((1, 1), lambda i: (0, 0)),
        ],
        out_specs=[pl.BlockSpec((BLK, 1), lambda i: (i, 0))],
        out_shape=[jax.ShapeDtypeStruct((NP, 1), jnp.float32)],
    )(oraw2, oden2, b2r, M2, Wh, bhr)


def kernel(x, edge_index, W1, att_src1, att_dst1, b1,
           W2, att_src2, att_dst2, b2, Wh, bh):
    f32 = jnp.float32
    x_p = jnp.pad(x, ((0, NP - N), (0, 0)))

    loop = jnp.arange(N, dtype=jnp.int32)
    padv = jnp.full((E_PAD - E_TOT,), PAD_NODE, jnp.int32)
    src = jnp.concatenate([edge_index[0], loop, padv]).reshape(E_PAD // KCH, KCH)
    dst = jnp.concatenate([edge_index[1], loop, padv]).reshape(E_PAD // KCH, KCH)

    ar = jnp.arange(D)
    amat1 = jnp.zeros((D, 16), f32)
    amat1 = amat1.at[ar, ar // 16].set(att_src1.reshape(D))
    amat1 = amat1.at[ar, ar // 16 + 8].set(att_dst1.reshape(D))
    amat2 = jnp.zeros((32, 16), f32)
    amat2 = amat2.at[:, 0].set(att_src2.reshape(32))
    amat2 = amat2.at[:, 8].set(att_dst2.reshape(32))
    M1 = jnp.zeros((16, D), f32).at[ar // 16, ar].set(1.0)
    M2 = jnp.zeros((16, 32), f32).at[0, :].set(1.0)

    h1, attT1, m1 = _tc1(x_p, W1, amat1)
    oraw1, oden1 = _sc_edge(128, 8)(src, dst, h1, attT1, m1)
    h2, attT2, m2 = _tc2(oraw1, oden1, b1.reshape(1, D), M1, W2, amat2)
    oraw2, oden2 = _sc_edge(32, 1)(src, dst, h2, attT2, m2)
    (y,) = _tc3(oraw2, oden2, b2.reshape(1, 32), M2, Wh, bh.reshape(1, 1))
    return y[:N]


# speedup vs baseline: 38.8001x; 1.2785x over previous
import functools

import jax
import jax.numpy as jnp
from jax import lax
from jax.experimental import pallas as pl
from jax.experimental.pallas import tpu as pltpu
from jax.experimental.pallas import tpu_sc as plsc

N = 10000
NP = 10240
D = 128
E = 320000
E_TOT = N + E
KCH = 88
NTILES = 32
TCH = 120
E_PAD = NTILES * TCH * KCH
PAD_NODE = 10200
BLK = 1024
ROWS_PER_SUB = NP // 16


def _lgather(v, idx):
    dn = lax.GatherDimensionNumbers(
        offset_dims=(), collapsed_slice_dims=(0,), start_index_map=(0,))
    return lax.gather(v, idx[:, None], dn, (1,),
                      mode=lax.GatherScatterMode.PROMISE_IN_BOUNDS)


def _elu(v):
    return jnp.where(v > 0, v, jnp.exp(jnp.minimum(v, 0.0)) - 1.0)


def _tc1_body(x_r, w_r, a_r, h_r, att_r, m_r, macc):
    i = pl.program_id(0)
    n = pl.num_programs(0)
    h = jnp.dot(x_r[...], w_r[...], preferred_element_type=jnp.float32)
    h_r[...] = h
    att = jnp.dot(h, a_r[...], preferred_element_type=jnp.float32)
    att_r[...] = att
    m = jnp.max(att, axis=0, keepdims=True)
    macc[...] = jnp.where(i == 0, m, jnp.maximum(macc[...], m))

    @pl.when(i == n - 1)
    def _():
        m_r[...] = macc[...]


def _tc1(x_p, W1, amat1):
    return pl.pallas_call(
        _tc1_body,
        grid=(NP // BLK,),
        in_specs=[
            pl.BlockSpec((BLK, D), lambda i: (i, 0)),
            pl.BlockSpec((D, D), lambda i: (0, 0)),
            pl.BlockSpec((D, 16), lambda i: (0, 0)),
        ],
        out_specs=[
            pl.BlockSpec((BLK, D), lambda i: (i, 0)),
            pl.BlockSpec((BLK, 16), lambda i: (i, 0)),
            pl.BlockSpec((1, 16), lambda i: (0, 0)),
        ],
        out_shape=[
            jax.ShapeDtypeStruct((NP, D), jnp.float32),
            jax.ShapeDtypeStruct((NP, 16), jnp.float32),
            jax.ShapeDtypeStruct((1, 16), jnp.float32),
        ],
        scratch_shapes=[pltpu.VMEM((1, 16), jnp.float32)],
    )(x_p, W1, amat1)


def _make_sc_edge(C, H):
    CPH = C // H
    mesh = plsc.VectorSubcoreMesh(core_axis_name="c", subcore_axis_name="s")

    @functools.partial(
        pl.kernel,
        out_type=(
            jax.ShapeDtypeStruct((2, NP, C), jnp.float32),
            jax.ShapeDtypeStruct((2, NP, 16), jnp.float32),
        ),
        mesh=mesh,
        compiler_params=pltpu.CompilerParams(use_tc_tiling_on_sc=False),
        scratch_types=[
            pltpu.VMEM((TCH, KCH), jnp.int32),
            pltpu.VMEM((TCH, KCH), jnp.int32),
            pltpu.VMEM((KCH, C), jnp.float32),
            pltpu.VMEM((KCH, 16), jnp.float32),
            pltpu.VMEM((KCH, 16), jnp.float32),
            pltpu.VMEM((KCH, 16), jnp.float32),
            pltpu.VMEM((16,), jnp.float32),
            pltpu.VMEM_SHARED((NP, C), jnp.float32),
            pltpu.VMEM_SHARED((NP, 16), jnp.float32),
        ],
    )
    def k(src_hbm, dst_hbm, h_hbm, att_hbm, m_hbm, oraw, oden,
          src2d, dst2d, rows, A, B, sden, gv, sout, sdsh):
        cid = lax.axis_index("c")
        sid = lax.axis_index("s")
        wid = sid * 2 + cid
        c0 = wid * TCH
        pltpu.sync_copy(src_hbm.at[pl.ds(c0, TCH)], src2d)
        pltpu.sync_copy(dst_hbm.at[pl.ds(c0, TCH)], dst2d)
        pltpu.sync_copy(m_hbm.at[0], gv)

        iota = lax.iota(jnp.int32, 16)
        idxshift = (iota + 8) & 15
        maskH = jnp.where(iota < H, 1.0, 0.0).astype(jnp.float32)
        m = gv[...]
        g = jnp.maximum(m + _lgather(m, idxshift), 0.0) * maskH
        hsel = [jnp.full((16,), h, jnp.int32) for h in range(H)]
        z16 = jnp.zeros((16,), jnp.float32)

        @plsc.parallel_loop(0, KCH, 1)
        def zrow(i):
            for kk in range(C // 16):
                rows[i, pl.ds(kk * 16, 16)] = z16
            sden[i] = z16
        r0 = sid * ROWS_PER_SUB
        for q in range(ROWS_PER_SUB // 80):
            pltpu.sync_copy(rows.at[pl.ds(0, 80)],
                            sout.at[pl.ds(r0 + q * 80, 80)])
            pltpu.sync_copy(sden.at[pl.ds(0, 80)],
                            sdsh.at[pl.ds(r0 + q * 80, 80)])
        plsc.subcore_barrier()

        def chunk(c, carry):
            pltpu.sync_copy(h_hbm.at[src2d.at[c]], rows)
            pltpu.sync_copy(att_hbm.at[src2d.at[c]], A)
            pltpu.sync_copy(att_hbm.at[dst2d.at[c]], B)

            @plsc.parallel_loop(0, KCH, 1, unroll=4)
            def edge(e):
                av = A[e]
                bv = B[e]
                al = av + _lgather(bv, idxshift)
                al = jnp.maximum(al, 0.2 * al)
                en = jnp.exp(al - g)
                sden[e] = en * maskH
                for h in range(H):
                    cs = _lgather(en, hsel[h])
                    for kk in range(CPH // 16):
                        sl = pl.ds(h * CPH + kk * 16, 16)
                        rows[e, sl] = rows[e, sl] * cs
            pltpu.sync_copy(rows, sout.at[dst2d.at[c]], add=True)
            pltpu.sync_copy(sden, sdsh.at[dst2d.at[c]], add=True)
            return carry
        lax.fori_loop(0, TCH, chunk, 0)
        plsc.subcore_barrier()

        pltpu.sync_copy(sout.at[pl.ds(r0, ROWS_PER_SUB)],
                        oraw.at[cid, pl.ds(r0, ROWS_PER_SUB)])
        pltpu.sync_copy(sdsh.at[pl.ds(r0, ROWS_PER_SUB)],
                        oden.at[cid, pl.ds(r0, ROWS_PER_SUB)])

    return k


_sc_cache = {}


def _sc_edge(C, H):
    if (C, H) not in _sc_cache:
        _sc_cache[(C, H)] = _make_sc_edge(C, H)
    return _sc_cache[(C, H)]


def _tc2_body(o_r, d_r, b_r, m1_r, w2_r, a2_r, h2_r, att_r, m_r, macc):
    i = pl.program_id(0)
    n = pl.num_programs(0)
    raw = o_r[0] + o_r[1]
    den = d_r[0] + d_r[1]
    denx = jnp.dot(den, m1_r[...], preferred_element_type=jnp.float32)
    x1 = _elu(raw / (denx + 1e-16) + b_r[...])
    h2 = jnp.dot(x1, w2_r[...], preferred_element_type=jnp.float32)
    h2_r[...] = h2
    att = jnp.dot(h2, a2_r[...], preferred_element_type=jnp.float32)
    att_r[...] = att
    m = jnp.max(att, axis=0, keepdims=True)
    macc[...] = jnp.where(i == 0, m, jnp.maximum(macc[...], m))

    @pl.when(i == n - 1)
    def _():
        m_r[...] = macc[...]


def _tc2(oraw, oden, b1r, M1, W2, amat2):
    return pl.pallas_call(
        _tc2_body,
        grid=(NP // BLK,),
        in_specs=[
            pl.BlockSpec((2, BLK, D), lambda i: (0, i, 0)),
            pl.BlockSpec((2, BLK, 16), lambda i: (0, i, 0)),
            pl.BlockSpec((1, D), lambda i: (0, 0)),
            pl.BlockSpec((16, D), lambda i: (0, 0)),
            pl.BlockSpec((D, 32), lambda i: (0, 0)),
            pl.BlockSpec((32, 16), lambda i: (0, 0)),
        ],
        out_specs=[
            pl.BlockSpec((BLK, 32), lambda i: (i, 0)),
            pl.BlockSpec((BLK, 16), lambda i: (i, 0)),
            pl.BlockSpec((1, 16), lambda i: (0, 0)),
        ],
        out_shape=[
            jax.ShapeDtypeStruct((NP, 32), jnp.float32),
            jax.ShapeDtypeStruct((NP, 16), jnp.float32),
            jax.ShapeDtypeStruct((1, 16), jnp.float32),
        ],
        scratch_shapes=[pltpu.VMEM((1, 16), jnp.float32)],
    )(oraw, oden, b1r, M1, W2, amat2)


def _tc3_body(o_r, d_r, b_r, m2_r, wh_r, bh_r, y_r):
    raw = o_r[0] + o_r[1]
    den = d_r[0] + d_r[1]
    denx = jnp.dot(den, m2_r[...], preferred_element_type=jnp.float32)
    x2 = _elu(raw / (denx + 1e-16) + b_r[...])
    y_r[...] = jnp.dot(x2, wh_r[...],
                       preferred_element_type=jnp.float32) + bh_r[...]


def _tc3(oraw2, oden2, b2r, M2, Wh, bhr):
    return pl.pallas_call(
        _tc3_body,
        grid=(NP // BLK,),
        in_specs=[
            pl.BlockSpec((2, BLK, 32), lambda i: (0, i, 0)),
            pl.BlockSpec((2, BLK, 16), lambda i: (0, i, 0)),
            pl.BlockSpec((1, 32), lambda i: (0, 0)),
            pl.BlockSpec((16, 32), lambda i: (0, 0)),
            pl.BlockSpec((32, 1), lambda i: (0, 0)),
            pl.BlockSpec((1, 1), lambda i: (0, 0)),
        ],
        out_specs=[pl.BlockSpec((BLK, 1), lambda i: (i, 0))],
        out_shape=[jax.ShapeDtypeStruct((NP, 1), jnp.float32)],
    )(oraw2, oden2, b2r, M2, Wh, bhr)


def kernel(x, edge_index, W1, att_src1, att_dst1, b1,
           W2, att_src2, att_dst2, b2, Wh, bh):
    f32 = jnp.float32
    x_p = jnp.pad(x, ((0, NP - N), (0, 0)))

    loop = jnp.arange(N, dtype=jnp.int32)
    padv = jnp.full((E_PAD - E_TOT,), PAD_NODE, jnp.int32)
    src = jnp.concatenate([edge_index[0], loop, padv]).reshape(E_PAD // KCH, KCH)
    dst = jnp.concatenate([edge_index[1], loop, padv]).reshape(E_PAD // KCH, KCH)

    ar = jnp.arange(D)
    amat1 = jnp.zeros((D, 16), f32)
    amat1 = amat1.at[ar, ar // 16].set(att_src1.reshape(D))
    amat1 = amat1.at[ar, ar // 16 + 8].set(att_dst1.reshape(D))
    amat2 = jnp.zeros((32, 16), f32)
    amat2 = amat2.at[:, 0].set(att_src2.reshape(32))
    amat2 = amat2.at[:, 8].set(att_dst2.reshape(32))
    M1 = jnp.zeros((16, D), f32).at[ar // 16, ar].set(1.0)
    M2 = jnp.zeros((16, 32), f32).at[0, :].set(1.0)

    h1, attT1, m1 = _tc1(x_p, W1, amat1)
    oraw1, oden1 = _sc_edge(128, 8)(src, dst, h1, attT1, m1)
    h2, attT2, m2 = _tc2(oraw1, oden1, b1.reshape(1, D), M1, W2, amat2)
    oraw2, oden2 = _sc_edge(32, 1)(src, dst, h2, attT2, m2)
    (y,) = _tc3(oraw2, oden2, b2.reshape(1, 32), M2, Wh, bh.reshape(1, 1))
    return y[:N]


# speedup vs baseline: 113.7903x; 2.9327x over previous
import functools

import jax
import jax.numpy as jnp
from jax import lax
from jax.experimental import pallas as pl
from jax.experimental.pallas import tpu as pltpu
from jax.experimental.pallas import tpu_sc as plsc

N = 10000
NP = 10240
D = 128
E = 320000
E_TOT = N + E
KCH = 72
NTILES = 32
TCH = 144
E_PAD = NTILES * TCH * KCH
PAD_NODE = 10200
BLK = 1024
ROWS_PER_SUB = NP // 16


def _lgather(v, idx):
    dn = lax.GatherDimensionNumbers(
        offset_dims=(), collapsed_slice_dims=(0,), start_index_map=(0,))
    return lax.gather(v, idx[:, None], dn, (1,),
                      mode=lax.GatherScatterMode.PROMISE_IN_BOUNDS)


def _elu(v):
    return jnp.where(v > 0, v, jnp.exp(jnp.minimum(v, 0.0)) - 1.0)


def _tc1_body(x_r, w_r, a_r, h_r, att_r, m_r, macc):
    i = pl.program_id(0)
    n = pl.num_programs(0)
    h = jnp.dot(x_r[...], w_r[...], preferred_element_type=jnp.float32)
    h_r[...] = h
    att = jnp.dot(h, a_r[...], preferred_element_type=jnp.float32)
    att_r[...] = att
    m = jnp.max(att, axis=0, keepdims=True)
    macc[...] = jnp.where(i == 0, m, jnp.maximum(macc[...], m))

    @pl.when(i == n - 1)
    def _():
        m_r[...] = macc[...]


def _tc1(x_p, W1, amat1):
    return pl.pallas_call(
        _tc1_body,
        grid=(NP // BLK,),
        in_specs=[
            pl.BlockSpec((BLK, D), lambda i: (i, 0)),
            pl.BlockSpec((D, D), lambda i: (0, 0)),
            pl.BlockSpec((D, 16), lambda i: (0, 0)),
        ],
        out_specs=[
            pl.BlockSpec((BLK, D), lambda i: (i, 0)),
            pl.BlockSpec((BLK, 16), lambda i: (i, 0)),
            pl.BlockSpec((1, 16), lambda i: (0, 0)),
        ],
        out_shape=[
            jax.ShapeDtypeStruct((NP, D), jnp.float32),
            jax.ShapeDtypeStruct((NP, 16), jnp.float32),
            jax.ShapeDtypeStruct((1, 16), jnp.float32),
        ],
        scratch_shapes=[pltpu.VMEM((1, 16), jnp.float32)],
    )(x_p, W1, amat1)


def _make_sc_edge(C, H):
    CPH = C // H
    mesh = plsc.VectorSubcoreMesh(core_axis_name="c", subcore_axis_name="s")

    @functools.partial(
        pl.kernel,
        out_type=(
            jax.ShapeDtypeStruct((2, NP, C), jnp.float32),
            jax.ShapeDtypeStruct((2, NP, 16), jnp.float32),
        ),
        mesh=mesh,
        compiler_params=pltpu.CompilerParams(use_tc_tiling_on_sc=False),
        scratch_types=[
            [pltpu.VMEM((2, KCH), jnp.int32) for _ in range(4)],
            [pltpu.VMEM((KCH, C), jnp.float32) for _ in range(3)],
            [pltpu.VMEM((KCH, 16), jnp.float32) for _ in range(3)],
            [pltpu.VMEM((KCH, 16), jnp.float32) for _ in range(3)],
            [pltpu.VMEM((KCH, 16), jnp.float32) for _ in range(3)],
            pltpu.VMEM((16,), jnp.float32),
            pltpu.VMEM_SHARED((NP, C), jnp.float32),
            pltpu.VMEM_SHARED((NP, 16), jnp.float32),
            [pltpu.SemaphoreType.DMA for _ in range(4)],
            [pltpu.SemaphoreType.DMA for _ in range(3)],
            [pltpu.SemaphoreType.DMA for _ in range(3)],
        ],
    )
    def k(eidx_hbm, h_hbm, att_hbm, m_hbm, oraw, oden,
          ibuf, rows3, A3, B3, sden3, gv, sout, sdsh, isem, gsem, ssem):
        cid = lax.axis_index("c")
        sid = lax.axis_index("s")
        wid = sid * 2 + cid
        c0 = wid * TCH
        pltpu.sync_copy(m_hbm.at[0], gv)

        iota = lax.iota(jnp.int32, 16)
        idxshift = (iota + 8) & 15
        maskH = jnp.where(iota < H, 1.0, 0.0).astype(jnp.float32)
        m = gv[...]
        g = jnp.maximum(m + _lgather(m, idxshift), 0.0) * maskH
        hsel = [jnp.full((16,), h, jnp.int32) for h in range(H)]
        z16 = jnp.zeros((16,), jnp.float32)

        rows0, sden0 = rows3[0], sden3[0]

        @plsc.parallel_loop(0, KCH, 1)
        def zrow(i):
            for kk in range(C // 16):
                rows0[i, pl.ds(kk * 16, 16)] = z16
            sden0[i] = z16
        r0 = sid * ROWS_PER_SUB
        for q in range(ROWS_PER_SUB // 64):
            pltpu.sync_copy(rows0.at[pl.ds(0, 64)],
                            sout.at[pl.ds(r0 + q * 64, 64)])
            pltpu.sync_copy(sden0.at[pl.ds(0, 64)],
                            sdsh.at[pl.ds(r0 + q * 64, 64)])
        plsc.subcore_barrier()

        def _fetch_idx(c, i):
            pltpu.async_copy(eidx_hbm.at[c0 + c], ibuf[i], isem[i])

        def _wait_idx(c, i):
            pltpu.make_async_copy(eidx_hbm.at[c0 + c], ibuf[i],
                                  isem[i]).wait()

        def _gather(i, b):
            pltpu.async_copy(h_hbm.at[ibuf[i].at[0]], rows3[b], gsem[b])
            pltpu.async_copy(att_hbm.at[ibuf[i].at[0]], A3[b], gsem[b])
            pltpu.async_copy(att_hbm.at[ibuf[i].at[1]], B3[b], gsem[b])

        def _wait_gather(i, b):
            pltpu.make_async_copy(h_hbm.at[ibuf[i].at[0]], rows3[b],
                                  gsem[b]).wait()
            pltpu.make_async_copy(att_hbm.at[ibuf[i].at[0]], A3[b],
                                  gsem[b]).wait()
            pltpu.make_async_copy(att_hbm.at[ibuf[i].at[1]], B3[b],
                                  gsem[b]).wait()

        def _scatter(i, b):
            pltpu.async_copy(rows3[b], sout.at[ibuf[i].at[1]], ssem[b],
                             add=True)
            pltpu.async_copy(sden3[b], sdsh.at[ibuf[i].at[1]], ssem[b],
                             add=True)

        def _wait_scatter(i, b):
            pltpu.make_async_copy(rows3[b], sout.at[ibuf[i].at[1]],
                                  ssem[b]).wait()
            pltpu.make_async_copy(sden3[b], sdsh.at[ibuf[i].at[1]],
                                  ssem[b]).wait()

        def _compute(b):
            A, B, rows, sden = A3[b], B3[b], rows3[b], sden3[b]

            @plsc.parallel_loop(0, KCH, 1, unroll=4)
            def edge(e):
                av = A[e]
                bv = B[e]
                al = av + _lgather(bv, idxshift)
                al = jnp.maximum(al, 0.2 * al)
                en = jnp.exp(al - g)
                sden[e] = en * maskH
                for h in range(H):
                    cs = _lgather(en, hsel[h])
                    for kk in range(CPH // 16):
                        sl = pl.ds(h * CPH + kk * 16, 16)
                        rows[e, sl] = rows[e, sl] * cs

        pltpu.sync_copy(eidx_hbm.at[c0], ibuf[0])
        _fetch_idx(1, 1)
        _gather(0, 0)

        NG = TCH // 12

        def group(gi, carry):
            for j in range(12):
                c = gi * 12 + j
                b = j % 3
                i4 = j % 4
                if j >= 2:
                    _wait_scatter((j - 2) % 4, (b + 1) % 3)
                else:
                    @pl.when(gi > 0)
                    def _():
                        _wait_scatter((j - 2) % 4, (b + 1) % 3)
                if j >= 10:
                    @pl.when(gi < NG - 1)
                    def _():
                        _fetch_idx(c + 2, (j + 2) % 4)
                else:
                    _fetch_idx(c + 2, (j + 2) % 4)
                if j == 11:
                    @pl.when(gi < NG - 1)
                    def _():
                        _wait_idx(c + 1, (j + 1) % 4)
                        _gather((j + 1) % 4, (b + 1) % 3)
                else:
                    _wait_idx(c + 1, (j + 1) % 4)
                    _gather((j + 1) % 4, (b + 1) % 3)
                _wait_gather(i4, b)
                _compute(b)
                _scatter(i4, b)
            return carry
        lax.fori_loop(0, NG, group, 0)
        _wait_scatter(2, 1)
        _wait_scatter(3, 2)
        plsc.subcore_barrier()

        pltpu.sync_copy(sout.at[pl.ds(r0, ROWS_PER_SUB)],
                        oraw.at[cid, pl.ds(r0, ROWS_PER_SUB)])
        pltpu.sync_copy(sdsh.at[pl.ds(r0, ROWS_PER_SUB)],
                        oden.at[cid, pl.ds(r0, ROWS_PER_SUB)])

    return k


_sc_cache = {}


def _sc_edge(C, H):
    if (C, H) not in _sc_cache:
        _sc_cache[(C, H)] = _make_sc_edge(C, H)
    return _sc_cache[(C, H)]


def _tc2_body(o_r, d_r, b_r, m1_r, w2_r, a2_r, h2_r, att_r, m_r, macc):
    i = pl.program_id(0)
    n = pl.num_programs(0)
    raw = o_r[0] + o_r[1]
    den = d_r[0] + d_r[1]
    denx = jnp.dot(den, m1_r[...], preferred_element_type=jnp.float32)
    x1 = _elu(raw / (denx + 1e-16) + b_r[...])
    h2 = jnp.dot(x1, w2_r[...], preferred_element_type=jnp.float32)
    h2_r[...] = h2
    att = jnp.dot(h2, a2_r[...], preferred_element_type=jnp.float32)
    att_r[...] = att
    m = jnp.max(att, axis=0, keepdims=True)
    macc[...] = jnp.where(i == 0, m, jnp.maximum(macc[...], m))

    @pl.when(i == n - 1)
    def _():
        m_r[...] = macc[...]


def _tc2(oraw, oden, b1r, M1, W2, amat2):
    return pl.pallas_call(
        _tc2_body,
        grid=(NP // BLK,),
        in_specs=[
            pl.BlockSpec((2, BLK, D), lambda i: (0, i, 0)),
            pl.BlockSpec((2, BLK, 16), lambda i: (0, i, 0)),
            pl.BlockSpec((1, D), lambda i: (0, 0)),
            pl.BlockSpec((16, D), lambda i: (0, 0)),
            pl.BlockSpec((D, 32), lambda i: (0, 0)),
            pl.BlockSpec((32, 16), lambda i: (0, 0)),
        ],
        out_specs=[
            pl.BlockSpec((BLK, 32), lambda i: (i, 0)),
            pl.BlockSpec((BLK, 16), lambda i: (i, 0)),
            pl.BlockSpec((1, 16), lambda i: (0, 0)),
        ],
        out_shape=[
            jax.ShapeDtypeStruct((NP, 32), jnp.float32),
            jax.ShapeDtypeStruct((NP, 16), jnp.float32),
            jax.ShapeDtypeStruct((1, 16), jnp.float32),
        ],
        scratch_shapes=[pltpu.VMEM((1, 16), jnp.float32)],
    )(oraw, oden, b1r, M1, W2, amat2)


def _tc3_body(o_r, d_r, b_r, m2_r, wh_r, bh_r, y_r):
    raw = o_r[0] + o_r[1]
    den = d_r[0] + d_r[1]
    denx = jnp.dot(den, m2_r[...], preferred_element_type=jnp.float32)
    x2 = _elu(raw / (denx + 1e-16) + b_r[...])
    y_r[...] = jnp.dot(x2, wh_r[...],
                       preferred_element_type=jnp.float32) + bh_r[...]


def _tc3(oraw2, oden2, b2r, M2, Wh, bhr):
    return pl.pallas_call(
        _tc3_body,
        grid=(NP // BLK,),
        in_specs=[
            pl.BlockSpec((2, BLK, 32), lambda i: (0, i, 0)),
            pl.BlockSpec((2, BLK, 16), lambda i: (0, i, 0)),
            pl.BlockSpec((1, 32), lambda i: (0, 0)),
            pl.BlockSpec((16, 32), lambda i: (0, 0)),
            pl.BlockSpec((32, 1), lambda i: (0, 0)),
            pl.BlockSpec((1, 1), lambda i: (0, 0)),
        ],
        out_specs=[pl.BlockSpec((BLK, 1), lambda i: (i, 0))],
        out_shape=[jax.ShapeDtypeStruct((NP, 1), jnp.float32)],
    )(oraw2, oden2, b2r, M2, Wh, bhr)


def kernel(x, edge_index, W1, att_src1, att_dst1, b1,
           W2, att_src2, att_dst2, b2, Wh, bh):
    f32 = jnp.float32
    x_p = jnp.pad(x, ((0, NP - N), (0, 0)))

    loop = jnp.arange(N, dtype=jnp.int32)
    padv = jnp.full((E_PAD - E_TOT,), PAD_NODE, jnp.int32)
    src = jnp.concatenate([edge_index[0], loop, padv]).reshape(E_PAD // KCH, KCH)
    dst = jnp.concatenate([edge_index[1], loop, padv]).reshape(E_PAD // KCH, KCH)
    eidx = jnp.stack([src, dst], axis=1)

    ar = jnp.arange(D)
    amat1 = jnp.zeros((D, 16), f32)
    amat1 = amat1.at[ar, ar // 16].set(att_src1.reshape(D))
    amat1 = amat1.at[ar, ar // 16 + 8].set(att_dst1.reshape(D))
    amat2 = jnp.zeros((32, 16), f32)
    amat2 = amat2.at[:, 0].set(att_src2.reshape(32))
    amat2 = amat2.at[:, 8].set(att_dst2.reshape(32))
    M1 = jnp.zeros((16, D), f32).at[ar // 16, ar].set(1.0)
    M2 = jnp.zeros((16, 32), f32).at[0, :].set(1.0)

    h1, attT1, m1 = _tc1(x_p, W1, amat1)
    oraw1, oden1 = _sc_edge(128, 8)(eidx, h1, attT1, m1)
    h2, attT2, m2 = _tc2(oraw1, oden1, b1.reshape(1, D), M1, W2, amat2)
    oraw2, oden2 = _sc_edge(32, 1)(eidx, h2, attT2, m2)
    (y,) = _tc3(oraw2, oden2, b2.reshape(1, 32), M2, Wh, bh.reshape(1, 1))
    return y[:N]
